# baseline (speedup 1.0000x reference)
import functools
import math

import jax
import jax.numpy as jnp
from jax import lax
from jax.experimental import pallas as pl
from jax.experimental.pallas import tpu as pltpu
from jax.experimental.pallas import tpu_sc as plsc

_EMB = 128
_SCALE = math.sqrt(float(_EMB))

_NUM_SUBCORES = 16


@functools.lru_cache(maxsize=None)
def _sc_gather(n_rows: int):
    n_cores = 1
    b_per_w = n_rows // (n_cores * _NUM_SUBCORES)
    mesh = plsc.VectorSubcoreMesh(
        core_axis_name="c", subcore_axis_name="s", num_cores=n_cores)

    @functools.partial(
        pl.kernel,
        out_type=jax.ShapeDtypeStruct((n_rows, _EMB), jnp.float32),
        mesh=mesh,
        scratch_types=[
            pltpu.VMEM((b_per_w,), jnp.int32),
            pltpu.VMEM((b_per_w, _EMB), jnp.float32),
            pltpu.SemaphoreType.DMA,
        ],
        compiler_params=pltpu.CompilerParams(use_tc_tiling_on_sc=True),
    )
    def gather(table_hbm, idx_hbm, out_hbm, idx_v, rows_v, sem):
        wid = lax.axis_index("s") * n_cores + lax.axis_index("c")
        base = wid * b_per_w
        pltpu.sync_copy(idx_hbm.at[pl.ds(base, b_per_w)], idx_v)
        pltpu.async_copy(table_hbm.at[idx_v], rows_v, sem).wait()
        pltpu.sync_copy(rows_v, out_hbm.at[pl.ds(base, b_per_w)])

    return gather


def _add_body(x_ref, enc_ref, o_ref):
    o_ref[...] = x_ref[...] + enc_ref[...] * _SCALE


_ROWS_PER_BLOCK = 32


@functools.lru_cache(maxsize=None)
def _tc_add(n_rows: int, tokens: int):
    r = _ROWS_PER_BLOCK
    return pl.pallas_call(
        _add_body,
        grid=(n_rows // r,),
        in_specs=[
            pl.BlockSpec((r, tokens, _EMB), lambda i: (i, 0, 0)),
            pl.BlockSpec((r, 1, _EMB), lambda i: (i, 0, 0)),
        ],
        out_specs=pl.BlockSpec((r, tokens, _EMB), lambda i: (i, 0, 0)),
        out_shape=jax.ShapeDtypeStruct((n_rows, tokens, _EMB), jnp.float32),
        compiler_params=pltpu.CompilerParams(
            vmem_limit_bytes=100 * 1024 * 1024),
    )


def kernel(x, blocks, emb_table):
    batch, num_blocks, tokens, emb = x.shape
    n_rows = batch * num_blocks
    idx = blocks.astype(jnp.int32).reshape(n_rows)
    enc = _sc_gather(n_rows)(emb_table, idx)
    out = _tc_add(n_rows, tokens)(
        x.reshape(n_rows, tokens, emb), enc.reshape(n_rows, 1, emb)
    )
    return out.reshape(batch, num_blocks * tokens, emb)
